# Initial kernel scaffold; baseline (speedup 1.0000x reference)
import jax
import jax.numpy as jnp
from jax.experimental import pallas as pl


def kernel(check_features, var_features, W_check_in, b_check_in, W_var_in, b_var_in, ln_c_gamma, ln_c_beta, ln_v_gamma, ln_v_beta, W_msg_vc, b_msg_vc, W_msg_cv, b_msg_cv, W_next_c, b_next_c, W_next_v, b_next_v, gru_c_Wx, gru_c_Wh, gru_c_bx, gru_c_bh, gru_v_Wx, gru_v_Wh, gru_v_bx, gru_v_bh, W_out, b_out, c_to_v_sources, c_to_v_targets, v_to_c_sources, v_to_c_targets):
    raise NotImplementedError("write your pallas kernel here")


# speedup vs baseline: 1.5678x; 1.5678x over previous
import functools

import jax
import jax.numpy as jnp
from jax import lax
from jax.experimental import pallas as pl
from jax.experimental.pallas import tpu as pltpu
from jax.experimental.pallas import tpu_sc as plsc

NC = 10000
NV = 50000
E = 320000
H = 128
L = 5

f32 = jnp.float32
i32 = jnp.int32

KA = 80
EWA = KA * 128
EA = 32 * EWA
KB = 160
EWB = KB * 128
EB = 16 * EWB
G = 4
DG = H // G
NCP = NC + 112
NVP = NV + 48
RPT_A = NCP // 16
RPT_B = NVP // 16
SIB = 16


def _elu(x):
    return jnp.where(x > 0, x, jnp.exp(jnp.minimum(x, 0.0)) - 1.0)


def _sigmoid(x):
    return 1.0 / (1.0 + jnp.exp(-x))


def _init_body(f_ref, w_ref, b_ref, g_ref, be_ref, o_ref):
    x = (f_ref[...] * 0.1) * w_ref[...] + b_ref[...]
    m = jnp.mean(x, axis=-1, keepdims=True)
    v = jnp.mean((x - m) * (x - m), axis=-1, keepdims=True)
    o_ref[...] = (x - m) / jnp.sqrt(v + 1e-6) * g_ref[...] + be_ref[...]


def _init_state(feat, w_in, b_in, ln_g, ln_b, n, blk):
    grid = (n // blk,)
    row = lambda r: (r, 0)
    zero = lambda r: (0, 0)
    return pl.pallas_call(
        _init_body,
        grid=grid,
        in_specs=[
            pl.BlockSpec((blk, 1), row),
            pl.BlockSpec((1, H), zero),
            pl.BlockSpec((1, H), zero),
            pl.BlockSpec((1, H), zero),
            pl.BlockSpec((1, H), zero),
        ],
        out_specs=pl.BlockSpec((blk, H), row),
        out_shape=jax.ShapeDtypeStruct((n, H), f32),
    )(feat, w_in.reshape(1, H), b_in.reshape(1, H),
      ln_g.reshape(1, H), ln_b.reshape(1, H))


def _msg_body(x_ref, w_ref, b_ref, o_ref):
    o_ref[...] = _elu(
        jnp.dot(x_ref[...], w_ref[...], preferred_element_type=f32)
        + b_ref[...])


def _msg_full(x, w, b, n, blk):
    return pl.pallas_call(
        _msg_body,
        grid=(n // blk,),
        in_specs=[
            pl.BlockSpec((blk, H), lambda r: (r, 0)),
            pl.BlockSpec((H, H), lambda r: (0, 0)),
            pl.BlockSpec((1, H), lambda r: (0, 0)),
        ],
        out_specs=pl.BlockSpec((blk, H), lambda r: (r, 0)),
        out_shape=jax.ShapeDtypeStruct((n, H), f32),
    )(x, w, b.reshape(1, H))


def _msg_grouped_body(x_ref, w_ref, b_ref, o_ref):
    o_ref[...] = _elu(
        jnp.dot(x_ref[...], w_ref[0], preferred_element_type=f32)
        + b_ref[0])


def _msg_grouped(x, w, b, n, blk):
    nb = n // blk
    wg = w.reshape(H, G, DG).transpose(1, 0, 2)
    bg = b.reshape(G, 1, DG)
    return pl.pallas_call(
        _msg_grouped_body,
        grid=(nb, G),
        in_specs=[
            pl.BlockSpec((blk, H), lambda r, g: (r, 0)),
            pl.BlockSpec((1, H, DG), lambda r, g: (g, 0, 0)),
            pl.BlockSpec((1, 1, DG), lambda r, g: (g, 0, 0)),
        ],
        out_specs=pl.BlockSpec((blk, DG), lambda r, g: (g * nb + r, 0)),
        out_shape=jax.ShapeDtypeStruct((G * n, DG), f32),
    )(x, wg, bg)


def _gru_update(s, upd, wx, wh, bx, bh):
    xm = jnp.dot(upd, wx, preferred_element_type=f32) + bx
    hm = jnp.dot(s, wh, preferred_element_type=f32) + bh
    z = _sigmoid(xm[:, :H] + hm[:, :H])
    r = _sigmoid(xm[:, H:2 * H] + hm[:, H:2 * H])
    hh = jnp.tanh(xm[:, 2 * H:] + r * hm[:, 2 * H:])
    return z * s + (1.0 - z) * hh


def _upd_c_body(s_ref, p0_ref, p1_ref, w1_ref, w2_ref, wx_ref, wh_ref,
                bn_ref, bx_ref, bh_ref, o_ref):
    s = s_ref[...]
    pooled = p0_ref[...] + p1_ref[...]
    upd = _elu(jnp.dot(s, w1_ref[...], preferred_element_type=f32)
               + jnp.dot(pooled, w2_ref[...], preferred_element_type=f32)
               + bn_ref[...])
    o_ref[...] = _gru_update(s, upd, wx_ref[...], wh_ref[...],
                             bx_ref[...], bh_ref[...])


def _update_c(cs, partials, w_next, b_next, wx, wh, bx, bh, blk):
    nb = NC // blk
    zero = lambda r: (0, 0)
    row = lambda r: (r, 0)
    return pl.pallas_call(
        _upd_c_body,
        grid=(nb,),
        in_specs=[
            pl.BlockSpec((blk, H), row),
            pl.BlockSpec((blk, H), row),
            pl.BlockSpec((blk, H), lambda r: (nb + r, 0)),
            pl.BlockSpec((H, H), zero),
            pl.BlockSpec((H, H), zero),
            pl.BlockSpec((H, 3 * H), zero),
            pl.BlockSpec((H, 3 * H), zero),
            pl.BlockSpec((1, H), zero),
            pl.BlockSpec((1, 3 * H), zero),
            pl.BlockSpec((1, 3 * H), zero),
        ],
        out_specs=pl.BlockSpec((blk, H), row),
        out_shape=jax.ShapeDtypeStruct((NC, H), f32),
    )(cs, partials, partials, w_next[:H], w_next[H:], wx, wh,
      b_next.reshape(1, H), bx.reshape(1, 3 * H), bh.reshape(1, 3 * H))


def _upd_v_body(s_ref, p0_ref, p1_ref, p2_ref, p3_ref, w1_ref, w2_ref,
                wx_ref, wh_ref, bn_ref, bx_ref, bh_ref, o_ref):
    s = s_ref[...]
    pooled = jnp.concatenate(
        [p0_ref[...], p1_ref[...], p2_ref[...], p3_ref[...]], axis=-1)
    upd = _elu(jnp.dot(s, w1_ref[...], preferred_element_type=f32)
               + jnp.dot(pooled, w2_ref[...], preferred_element_type=f32)
               + bn_ref[...])
    o_ref[...] = _gru_update(s, upd, wx_ref[...], wh_ref[...],
                             bx_ref[...], bh_ref[...])


def _upd_v_last_body(s_ref, p0_ref, p1_ref, p2_ref, p3_ref, w1_ref, w2_ref,
                     wx_ref, wh_ref, bn_ref, bx_ref, bh_ref, wo_ref, bo_ref,
                     o_ref):
    s = s_ref[...]
    pooled = jnp.concatenate(
        [p0_ref[...], p1_ref[...], p2_ref[...], p3_ref[...]], axis=-1)
    upd = _elu(jnp.dot(s, w1_ref[...], preferred_element_type=f32)
               + jnp.dot(pooled, w2_ref[...], preferred_element_type=f32)
               + bn_ref[...])
    sn = _gru_update(s, upd, wx_ref[...], wh_ref[...], bx_ref[...], bh_ref[...])
    o_ref[...] = jnp.dot(sn, wo_ref[...], preferred_element_type=f32) + bo_ref[...]


def _update_v(vs, pooled_flat, w_next, b_next, wx, wh, bx, bh, blk,
              w_out=None, b_out=None):
    nb = NV // blk
    zero = lambda r: (0, 0)
    row = lambda r: (r, 0)
    last = w_out is not None
    in_specs = [pl.BlockSpec((blk, H), row)]
    in_specs += [pl.BlockSpec((blk, DG), functools.partial(
        lambda g, r: (g * nb + r, 0), g)) for g in range(G)]
    in_specs += [
        pl.BlockSpec((H, H), zero),
        pl.BlockSpec((H, H), zero),
        pl.BlockSpec((H, 3 * H), zero),
        pl.BlockSpec((H, 3 * H), zero),
        pl.BlockSpec((1, H), zero),
        pl.BlockSpec((1, 3 * H), zero),
        pl.BlockSpec((1, 3 * H), zero),
    ]
    args = [vs, pooled_flat, pooled_flat, pooled_flat, pooled_flat,
            w_next[:H], w_next[H:], wx, wh, b_next.reshape(1, H),
            bx.reshape(1, 3 * H), bh.reshape(1, 3 * H)]
    if last:
        in_specs += [pl.BlockSpec((H, 1), zero), pl.BlockSpec((1, 1), zero)]
        args += [w_out, b_out.reshape(1, 1)]
        return pl.pallas_call(
            _upd_v_last_body,
            grid=(nb,),
            in_specs=in_specs,
            out_specs=pl.BlockSpec((blk, 1), row),
            out_shape=jax.ShapeDtypeStruct((NV, 1), f32),
        )(*args)
    return pl.pallas_call(
        _upd_v_body,
        grid=(nb,),
        in_specs=in_specs,
        out_specs=pl.BlockSpec((blk, H), row),
        out_shape=jax.ShapeDtypeStruct((NV, H), f32),
    )(*args)


def _mesh():
    return plsc.VectorSubcoreMesh(core_axis_name="c", subcore_axis_name="s")


def _sc_seg_sum_c(mv_hbm, sidx_hbm, tidx_hbm, zeros_hbm, out_hbm,
                  sidx_v, tidx_v, rows_v, acc_sh, sem):
    c = lax.axis_index("c")
    s = lax.axis_index("s")
    w = c * 16 + s
    pltpu.sync_copy(zeros_hbm, acc_sh.at[pl.ds(s * RPT_A, RPT_A)])
    pltpu.sync_copy(sidx_hbm.at[pl.ds(w * KA, KA)], sidx_v)
    pltpu.sync_copy(tidx_hbm.at[pl.ds(w * KA, KA)], tidx_v)
    plsc.subcore_barrier()

    def body(j, carry):
        pltpu.async_copy(mv_hbm.at[sidx_v.at[j]], rows_v, sem).wait()
        pltpu.sync_copy(rows_v, acc_sh.at[tidx_v.at[j]], add=True)
        return carry

    lax.fori_loop(0, KA, body, 0, unroll=False)
    plsc.subcore_barrier()
    @pl.when(s < 15)
    def _():
        pltpu.sync_copy(acc_sh.at[pl.ds(s * RPT_A, RPT_A)],
                        out_hbm.at[pl.ds(c * NC + s * RPT_A, RPT_A)])

    @pl.when(s == 15)
    def _():
        pltpu.sync_copy(
            acc_sh.at[pl.ds(15 * RPT_A, NC - 15 * RPT_A)],
            out_hbm.at[pl.ds(c * NC + 15 * RPT_A, NC - 15 * RPT_A)])


def _seg_sum_c(mv, sidx, tidx, zeros):
    k = pl.kernel(
        _sc_seg_sum_c,
        out_type=jax.ShapeDtypeStruct((2 * NC, H), f32),
        mesh=_mesh(),
        scratch_types=[
            pltpu.VMEM((KA, 128), i32),
            pltpu.VMEM((KA, 128), i32),
            pltpu.VMEM((128, H), f32),
            pltpu.VMEM_SHARED((NCP, H), f32),
            pltpu.SemaphoreType.DMA,
        ],
    )
    return k(mv, sidx, tidx, zeros)


def _sc_seg_sum_v(mc_hbm, sidx_hbm, tidx_hbm, zeros_hbm, out_hbm,
                  sidx_v, tidx_v, rows_v, acc_sh, sem):
    c = lax.axis_index("c")
    s = lax.axis_index("s")
    for p in range(2):
        g = c * 2 + p
        pltpu.sync_copy(zeros_hbm, acc_sh.at[pl.ds(s * RPT_B, RPT_B)])
        plsc.subcore_barrier()

        def blk_body(b, carry):
            pltpu.sync_copy(
                sidx_hbm.at[pl.ds((g * 16 + s) * KB + b * SIB, SIB)], sidx_v)
            pltpu.sync_copy(
                tidx_hbm.at[pl.ds(s * KB + b * SIB, SIB)], tidx_v)

            def body(j, carry2):
                pltpu.async_copy(mc_hbm.at[sidx_v.at[j]], rows_v, sem).wait()
                pltpu.sync_copy(rows_v, acc_sh.at[tidx_v.at[j]], add=True)
                return carry2

            lax.fori_loop(0, SIB, body, 0, unroll=False)
            return carry

        lax.fori_loop(0, KB // SIB, blk_body, 0, unroll=False)
        plsc.subcore_barrier()
        @pl.when(s < 15)
        def _():
            pltpu.sync_copy(acc_sh.at[pl.ds(s * RPT_B, RPT_B)],
                            out_hbm.at[pl.ds(g * NV + s * RPT_B, RPT_B)])

        @pl.when(s == 15)
        def _():
            pltpu.sync_copy(
                acc_sh.at[pl.ds(15 * RPT_B, NV - 15 * RPT_B)],
                out_hbm.at[pl.ds(g * NV + 15 * RPT_B, NV - 15 * RPT_B)])
        plsc.subcore_barrier()


def _seg_sum_v(mc, sidx, tidx, zeros):
    k = pl.kernel(
        _sc_seg_sum_v,
        out_type=jax.ShapeDtypeStruct((G * NV, DG), f32),
        mesh=_mesh(),
        compiler_params=pltpu.CompilerParams(use_tc_tiling_on_sc=False),
        scratch_types=[
            pltpu.VMEM((SIB, 128), i32),
            pltpu.VMEM((SIB, 128), i32),
            pltpu.VMEM((128, DG), f32),
            pltpu.VMEM_SHARED((NVP, DG), f32),
            pltpu.SemaphoreType.DMA,
        ],
    )
    return k(mc, sidx, tidx, zeros)


def kernel(check_features, var_features, W_check_in, b_check_in, W_var_in,
           b_var_in, ln_c_gamma, ln_c_beta, ln_v_gamma, ln_v_beta,
           W_msg_vc, b_msg_vc, W_msg_cv, b_msg_cv,
           W_next_c, b_next_c, W_next_v, b_next_v,
           gru_c_Wx, gru_c_Wh, gru_c_bx, gru_c_bh,
           gru_v_Wx, gru_v_Wh, gru_v_bx, gru_v_bh,
           W_out, b_out,
           c_to_v_sources, c_to_v_targets, v_to_c_sources, v_to_c_targets):
    sA = jnp.concatenate(
        [v_to_c_sources.astype(i32), jnp.zeros((EA - E,), i32)]
    ).reshape(32 * KA, 128)
    tA = jnp.concatenate(
        [v_to_c_targets.astype(i32), jnp.full((EA - E,), NC, i32)]
    ).reshape(32 * KA, 128)
    sB0 = jnp.concatenate(
        [c_to_v_sources.astype(i32), jnp.zeros((EB - E,), i32)])
    sB = (sB0[None, :] + (jnp.arange(G, dtype=i32) * NC)[:, None]
          ).reshape(G * 16 * KB, 128)
    tB = jnp.concatenate(
        [c_to_v_targets.astype(i32), jnp.full((EB - E,), NV, i32)]
    ).reshape(16 * KB, 128)
    zeros_c = jnp.zeros((RPT_A, H), f32)
    zeros_v = jnp.zeros((RPT_B, DG), f32)

    cs = _init_state(check_features, W_check_in, b_check_in,
                     ln_c_gamma, ln_c_beta, NC, 400)
    vs = _init_state(var_features, W_var_in, b_var_in,
                     ln_v_gamma, ln_v_beta, NV, 400)

    pred = None
    for r in range(L):
        last = r == L - 1
        mc = _msg_grouped(cs, W_msg_cv, b_msg_cv, NC, 400)
        pooled_v = _seg_sum_v(mc, sB, tB, zeros_v)
        if not last:
            mv = _msg_full(vs, W_msg_vc, b_msg_vc, NV, 400)
            pooled_c = _seg_sum_c(mv, sA, tA, zeros_c)
            cs = _update_c(cs, pooled_c, W_next_c, b_next_c,
                           gru_c_Wx, gru_c_Wh, gru_c_bx, gru_c_bh, 400)
            vs = _update_v(vs, pooled_v, W_next_v, b_next_v,
                           gru_v_Wx, gru_v_Wh, gru_v_bx, gru_v_bh, 400)
        else:
            pred = _update_v(vs, pooled_v, W_next_v, b_next_v,
                             gru_v_Wx, gru_v_Wh, gru_v_bx, gru_v_bh, 400,
                             w_out=W_out, b_out=b_out)
    return jnp.squeeze(pred, axis=-1)
